# baseline (speedup 1.0000x reference)
import functools

import jax
import jax.numpy as jnp
from jax import lax
from jax.experimental import pallas as pl
from jax.experimental.pallas import tpu as pltpu
from jax.experimental.pallas import tpu_sc as plsc

B = 16384
V_ITEM = 1000001
V_SPARSE = 100001
D_ITEM = 64
D_SPARSE = 32
MM_DIM = 128
D_MM = 32
DNN_HID = 256
HID_OUT = 128

_NC = 2
_NS = 16
_NW = _NC * _NS
_BPW = B // _NW
_JV = _BPW // 16
_CHUNK = 128

_BLK = 1024
_GRID = B // _BLK


_TABLES = (
    dict(d=D_ITEM, v=V_ITEM),
    dict(d=D_SPARSE, v=V_SPARSE),
    dict(d=D_SPARSE, v=V_SPARSE),
    dict(d=D_SPARSE, v=V_SPARSE),
    dict(d=D_SPARSE, v=V_SPARSE),
)


def _sc_gather_body(seq_ref, cate_ref, brand_ref, shop_ref, tag_ref,
                    t_item, t_cate, t_brand, t_shop, t_tag,
                    o_item, o_cate, o_brand, o_shop, o_tag,
                    idx_v, offs_v, gath_v, sem):
    wid = lax.axis_index("s") * _NC + lax.axis_index("c")
    base = wid * _BPW
    in_refs = (seq_ref, cate_ref, brand_ref, shop_ref, tag_ref)
    tabs = (t_item, t_cate, t_brand, t_shop, t_tag)
    outs = (o_item, o_cate, o_brand, o_shop, o_tag)

    iota16 = lax.iota(jnp.int32, 16)

    for t, spec in enumerate(_TABLES):
        d_t, v_t = spec["d"], spec["v"]
        n_el = _BPW * d_t
        ks = [d * v_t for d in range(d_t)]
        tab_flat = tabs[t]
        iota_d = iota16 * d_t

        pltpu.sync_copy(in_refs[t].at[pl.ds(base, _BPW)], idx_v)

        def offs_body(j, carry, *, d_t=d_t, ks=ks, iota_d=iota_d):
            bse = idx_v[pl.ds(j * 16, 16)]
            st0 = j * (16 * d_t)
            for d in range(d_t):
                plsc.store_scatter(offs_v, [iota_d + (st0 + d)], bse + ks[d])
            return carry

        lax.fori_loop(0, _JV, offs_body, 0, unroll=False)

        nch = n_el // _CHUNK

        def dma_body(k, carry, *, tab_flat=tab_flat):
            pltpu.async_copy(
                tab_flat.at[offs_v.at[pl.ds(k * _CHUNK, _CHUNK)]],
                gath_v.at[pl.ds(k * _CHUNK, _CHUNK)],
                sem)
            return carry

        lax.fori_loop(0, nch, dma_body, 0, unroll=False)
        pltpu.make_async_copy(
            tab_flat.at[pl.ds(0, n_el)], gath_v.at[pl.ds(0, n_el)], sem
        ).wait()
        pltpu.sync_copy(gath_v.at[pl.ds(0, n_el)],
                        outs[t].at[pl.ds(base * d_t, n_el)])


@jax.jit
def _sc_gather(seq_id, cate_id, brand_id, shop_id, tag_id,
               t_item, t_cate, t_brand, t_shop, t_tag):
    mesh = plsc.VectorSubcoreMesh(core_axis_name="c", subcore_axis_name="s")
    f32 = jnp.float32
    out_type = [
        jax.ShapeDtypeStruct((B * D_ITEM,), f32),
        jax.ShapeDtypeStruct((B * D_SPARSE,), f32),
        jax.ShapeDtypeStruct((B * D_SPARSE,), f32),
        jax.ShapeDtypeStruct((B * D_SPARSE,), f32),
        jax.ShapeDtypeStruct((B * D_SPARSE,), f32),
    ]
    scratch = [
        pltpu.VMEM((_BPW,), jnp.int32),
        pltpu.VMEM((_BPW * D_ITEM,), jnp.int32),
        pltpu.VMEM((_BPW * D_ITEM,), f32),
        pltpu.SemaphoreType.DMA,
    ]
    return pl.kernel(
        _sc_gather_body,
        out_type=out_type,
        mesh=mesh,
        scratch_types=scratch,
        compiler_params=pltpu.CompilerParams(
            disable_bounds_checks=True, needs_layout_passes=False),
    )(seq_id, cate_id, brand_id, shop_id, tag_id,
      t_item, t_cate, t_brand, t_shop, t_tag)


def _mlp_body(gi, gc, gb, gs, gt, dns, mm,
              mmW, mmb, w1i, w1c, w1b, w1s, w1t, w1d, w1m, b1, w2, b2,
              out):
    f32 = jnp.float32
    acc = jnp.dot(gi[...], w1i[...], preferred_element_type=f32)
    acc += jnp.dot(gc[...], w1c[...], preferred_element_type=f32)
    acc += jnp.dot(gb[...], w1b[...], preferred_element_type=f32)
    acc += jnp.dot(gs[...], w1s[...], preferred_element_type=f32)
    acc += jnp.dot(gt[...], w1t[...], preferred_element_type=f32)
    acc += jnp.dot(dns[...], w1d[...], preferred_element_type=f32)
    mmp = jnp.dot(mm[...], mmW[...], preferred_element_type=f32) + mmb[...]
    acc += jnp.dot(mmp, w1m[...], preferred_element_type=f32)
    acc += b1[...]
    h = jnp.maximum(acc, 0.0)
    out[...] = jnp.dot(h, w2[...], preferred_element_type=f32) + b2[...]


def _full(shape):
    return pl.BlockSpec(shape, lambda i: (0, 0))


def _mlp(gi, gc, gb, gs, gt, dns, mm, mmW, mmb,
         w1i, w1c, w1b, w1s, w1t, w1d, w1m, b1, w2, b2):
    blk = lambda d: pl.BlockSpec((_BLK, d), lambda i: (i, 0))
    in_specs = [
        blk(D_ITEM), blk(D_SPARSE), blk(D_SPARSE), blk(D_SPARSE), blk(D_SPARSE),
        blk(3), blk(MM_DIM),
        _full((MM_DIM, D_MM)), _full((1, D_MM)),
        _full((D_ITEM, DNN_HID)),
        _full((D_SPARSE, DNN_HID)), _full((D_SPARSE, DNN_HID)),
        _full((D_SPARSE, DNN_HID)), _full((D_SPARSE, DNN_HID)),
        _full((3, DNN_HID)), _full((D_MM, DNN_HID)),
        _full((1, DNN_HID)),
        _full((DNN_HID, HID_OUT)), _full((1, HID_OUT)),
    ]
    return pl.pallas_call(
        _mlp_body,
        grid=(_GRID,),
        in_specs=in_specs,
        out_specs=pl.BlockSpec((_BLK, HID_OUT), lambda i: (i, 0)),
        out_shape=jax.ShapeDtypeStruct((B, HID_OUT), jnp.float32),
        compiler_params=pltpu.CompilerParams(
            dimension_semantics=("arbitrary",)),
    )(gi, gc, gb, gs, gt, dns, mm, mmW, mmb,
      w1i, w1c, w1b, w1s, w1t, w1d, w1m, b1, w2, b2)


def kernel(seq_id, cate_id, brand_id, shop_id, tag_id,
           dense_0, dense_1, dense_2, mm_emb_0,
           emb_item, emb_cate, emb_brand, emb_shop, emb_tag,
           mm_W, mm_b, W1, b1, W2, b2):
    i32 = jnp.int32
    gi_f, gc_f, gb_f, gs_f, gt_f = _sc_gather(
        seq_id.astype(i32), cate_id.astype(i32), brand_id.astype(i32),
        shop_id.astype(i32), tag_id.astype(i32),
        jnp.ravel(emb_item.T), jnp.ravel(emb_cate.T), jnp.ravel(emb_brand.T),
        jnp.ravel(emb_shop.T), jnp.ravel(emb_tag.T))
    gi = gi_f.reshape(B, D_ITEM)
    gc = gc_f.reshape(B, D_SPARSE)
    gb = gb_f.reshape(B, D_SPARSE)
    gs = gs_f.reshape(B, D_SPARSE)
    gt = gt_f.reshape(B, D_SPARSE)

    dns = jnp.stack([dense_0, dense_1, dense_2], axis=1)
    w1i = W1[:D_ITEM]
    o = D_ITEM
    w1c = W1[o:o + D_SPARSE]; o += D_SPARSE
    w1b = W1[o:o + D_SPARSE]; o += D_SPARSE
    w1s = W1[o:o + D_SPARSE]; o += D_SPARSE
    w1t = W1[o:o + D_SPARSE]; o += D_SPARSE
    w1d = W1[o:o + 3]; o += 3
    w1m = W1[o:o + D_MM]

    return _mlp(gi, gc, gb, gs, gt, dns, mm_emb_0,
                mm_W, mm_b.reshape(1, -1),
                w1i, w1c, w1b, w1s, w1t, w1d, w1m,
                b1.reshape(1, -1), W2, b2.reshape(1, -1))


# speedup vs baseline: 2.3764x; 2.3764x over previous
import functools

import jax
import jax.numpy as jnp
from jax import lax
from jax.experimental import pallas as pl
from jax.experimental.pallas import tpu as pltpu
from jax.experimental.pallas import tpu_sc as plsc

B = 16384
D_ITEM = 64
D_SPARSE = 32
W_ITEM = D_ITEM // 2
W_SPARSE = D_SPARSE // 2
MM_DIM = 128
D_MM = 32
DNN_HID = 256
HID_OUT = 128

_NC = 2
_NS = 16
_NW = _NC * _NS
_BPW = B // _NW
_CHUNK = 128
_NCHUNK = _BPW // _CHUNK

_BLK = 1024
_GRID = B // _BLK


def _sc_gather_body(seq_ref, cate_ref, brand_ref, shop_ref, tag_ref,
                    t_item, t_cate, t_brand, t_shop, t_tag,
                    o_item, o_cate, o_brand, o_shop, o_tag,
                    idx0, idx1, idx2, idx3, idx4,
                    r0, r1, r2, r3, r4,
                    s0, s1, s2, s3, s4):
    wid = lax.axis_index("s") * _NC + lax.axis_index("c")
    base = wid * _BPW
    idx_refs = (idx0, idx1, idx2, idx3, idx4)
    in_refs = (seq_ref, cate_ref, brand_ref, shop_ref, tag_ref)
    tabs = (t_item, t_cate, t_brand, t_shop, t_tag)
    rows = (r0, r1, r2, r3, r4)
    outs = (o_item, o_cate, o_brand, o_shop, o_tag)
    sems = (s0, s1, s2, s3, s4)

    for i in range(5):
        pltpu.sync_copy(in_refs[i].at[pl.ds(base, _BPW)], idx_refs[i])

    handles = []
    for i in range(5):
        per_tab = []
        for j in range(_NCHUNK):
            h = pltpu.async_copy(
                tabs[i].at[idx_refs[i].at[pl.ds(j * _CHUNK, _CHUNK)]],
                rows[i].at[pl.ds(j * _CHUNK, _CHUNK)],
                sems[i])
            per_tab.append(h)
        handles.append(per_tab)

    for i in range(5):
        for h in handles[i]:
            h.wait()
        pltpu.sync_copy(rows[i], outs[i].at[pl.ds(base, _BPW)])


@jax.jit
def _sc_gather(seq_id, cate_id, brand_id, shop_id, tag_id,
               t_item, t_cate, t_brand, t_shop, t_tag):
    mesh = plsc.VectorSubcoreMesh(core_axis_name="c", subcore_axis_name="s")
    u32 = jnp.uint32
    out_type = [
        jax.ShapeDtypeStruct((B, W_ITEM), u32),
        jax.ShapeDtypeStruct((B, W_SPARSE), u32),
        jax.ShapeDtypeStruct((B, W_SPARSE), u32),
        jax.ShapeDtypeStruct((B, W_SPARSE), u32),
        jax.ShapeDtypeStruct((B, W_SPARSE), u32),
    ]
    scratch = (
        [pltpu.VMEM((_BPW,), jnp.int32) for _ in range(5)]
        + [pltpu.VMEM((_BPW, W_ITEM), u32)]
        + [pltpu.VMEM((_BPW, W_SPARSE), u32) for _ in range(4)]
        + [pltpu.SemaphoreType.DMA for _ in range(5)]
    )
    return pl.kernel(
        _sc_gather_body,
        out_type=out_type,
        mesh=mesh,
        scratch_types=scratch,
        compiler_params=pltpu.CompilerParams(use_tc_tiling_on_sc=False),
    )(seq_id, cate_id, brand_id, shop_id, tag_id,
      t_item, t_cate, t_brand, t_shop, t_tag)


def _mlp_body(gi, gc, gb, gs, gt, dns, mm,
              mmW, mmb, w1i, w1c, w1b, w1s, w1t, w1d, w1m, b1, w2, b2,
              out):
    f32 = jnp.float32
    acc = jnp.dot(gi[...], w1i[...], preferred_element_type=f32)
    acc += jnp.dot(gc[...], w1c[...], preferred_element_type=f32)
    acc += jnp.dot(gb[...], w1b[...], preferred_element_type=f32)
    acc += jnp.dot(gs[...], w1s[...], preferred_element_type=f32)
    acc += jnp.dot(gt[...], w1t[...], preferred_element_type=f32)
    acc += jnp.dot(dns[...], w1d[...], preferred_element_type=f32)
    mmp = jnp.dot(mm[...], mmW[...], preferred_element_type=f32) + mmb[...]
    acc += jnp.dot(mmp, w1m[...], preferred_element_type=f32)
    acc += b1[...]
    h = jnp.maximum(acc, 0.0)
    out[...] = jnp.dot(h, w2[...], preferred_element_type=f32) + b2[...]


def _full(shape):
    return pl.BlockSpec(shape, lambda i: (0, 0))


def _mlp(gi, gc, gb, gs, gt, dns, mm, mmW, mmb,
         w1i, w1c, w1b, w1s, w1t, w1d, w1m, b1, w2, b2):
    blk = lambda d: pl.BlockSpec((_BLK, d), lambda i: (i, 0))
    in_specs = [
        blk(D_ITEM), blk(D_SPARSE), blk(D_SPARSE), blk(D_SPARSE), blk(D_SPARSE),
        blk(3), blk(MM_DIM),
        _full((MM_DIM, D_MM)), _full((1, D_MM)),
        _full((D_ITEM, DNN_HID)),
        _full((D_SPARSE, DNN_HID)), _full((D_SPARSE, DNN_HID)),
        _full((D_SPARSE, DNN_HID)), _full((D_SPARSE, DNN_HID)),
        _full((3, DNN_HID)), _full((D_MM, DNN_HID)),
        _full((1, DNN_HID)),
        _full((DNN_HID, HID_OUT)), _full((1, HID_OUT)),
    ]
    return pl.pallas_call(
        _mlp_body,
        grid=(_GRID,),
        in_specs=in_specs,
        out_specs=pl.BlockSpec((_BLK, HID_OUT), lambda i: (i, 0)),
        out_shape=jax.ShapeDtypeStruct((B, HID_OUT), jnp.float32),
        compiler_params=pltpu.CompilerParams(
            dimension_semantics=("arbitrary",)),
    )(gi, gc, gb, gs, gt, dns, mm, mmW, mmb,
      w1i, w1c, w1b, w1s, w1t, w1d, w1m, b1, w2, b2)


def _pack(table):
    tb = table.astype(jnp.bfloat16)
    return jax.lax.bitcast_convert_type(
        tb.reshape(table.shape[0], table.shape[1] // 2, 2), jnp.uint32)


def _unpack(g):
    return jax.lax.bitcast_convert_type(g, jnp.bfloat16).reshape(g.shape[0], -1)


def kernel(seq_id, cate_id, brand_id, shop_id, tag_id,
           dense_0, dense_1, dense_2, mm_emb_0,
           emb_item, emb_cate, emb_brand, emb_shop, emb_tag,
           mm_W, mm_b, W1, b1, W2, b2):
    i32 = jnp.int32
    gi_u, gc_u, gb_u, gs_u, gt_u = _sc_gather(
        seq_id.astype(i32), cate_id.astype(i32), brand_id.astype(i32),
        shop_id.astype(i32), tag_id.astype(i32),
        _pack(emb_item), _pack(emb_cate), _pack(emb_brand),
        _pack(emb_shop), _pack(emb_tag))
    gi = _unpack(gi_u)
    gc = _unpack(gc_u)
    gb = _unpack(gb_u)
    gs = _unpack(gs_u)
    gt = _unpack(gt_u)

    dns = jnp.stack([dense_0, dense_1, dense_2], axis=1)
    w1i = W1[:D_ITEM]
    o = D_ITEM
    w1c = W1[o:o + D_SPARSE]; o += D_SPARSE
    w1b = W1[o:o + D_SPARSE]; o += D_SPARSE
    w1s = W1[o:o + D_SPARSE]; o += D_SPARSE
    w1t = W1[o:o + D_SPARSE]; o += D_SPARSE
    w1d = W1[o:o + 3]; o += 3
    w1m = W1[o:o + D_MM]

    return _mlp(gi, gc, gb, gs, gt, dns, mm_emb_0,
                mm_W, mm_b.reshape(1, -1),
                w1i, w1c, w1b, w1s, w1t, w1d, w1m,
                b1.reshape(1, -1), W2, b2.reshape(1, -1))
